# baseline (speedup 1.0000x reference)
import functools

import jax
import jax.numpy as jnp
from jax import lax
from jax.experimental import pallas as pl
from jax.experimental.pallas import tpu as pltpu
from jax.experimental.pallas import tpu_sc as plsc

DIM = 128
NUM_WORKERS = 32
IDXC = 128
GPB = 2
NBUF = 2

CHUNK = IDXC * GPB


def _make_gather(b_flat: int):
    b_per_w = b_flat // NUM_WORKERS
    nidx = b_per_w // IDXC
    nvisits = b_per_w // CHUNK
    ngroups = nvisits // NBUF
    mesh = plsc.VectorSubcoreMesh(core_axis_name="c", subcore_axis_name="s")

    @functools.partial(
        pl.kernel,
        mesh=mesh,
        out_type=jax.ShapeDtypeStruct((b_flat, DIM), jnp.float32),
        scratch_types=(
            [pltpu.VMEM((nidx, IDXC), jnp.int32)]
            + [pltpu.VMEM((CHUNK, DIM), jnp.float32) for _ in range(NBUF)]
            + [pltpu.SemaphoreType.DMA for _ in range(2 * NBUF)]
        ),
    )
    def gather_kernel(idx_hbm, table_hbm, out_hbm, idx_v, *rest):
        rows = rest[:NBUF]
        gsem = rest[NBUF:2 * NBUF]
        ssem = rest[2 * NBUF:]
        wid = lax.axis_index("s") * 2 + lax.axis_index("c")
        base_row = wid * b_per_w

        pltpu.sync_copy(idx_hbm.at[wid], idx_v)

        def fill(v, b):
            for j in range(GPB):
                pltpu.async_copy(
                    table_hbm.at[idx_v.at[v * GPB + j]],
                    rows[b].at[pl.ds(j * IDXC, IDXC)],
                    gsem[b],
                )

        def drain_fill(v, b):
            for j in range(GPB):
                pltpu.make_async_copy(
                    table_hbm.at[idx_v.at[v * GPB + j]],
                    rows[b].at[pl.ds(j * IDXC, IDXC)],
                    gsem[b],
                ).wait()

        for b in range(NBUF):
            fill(b, b)

        def body(group, carry):
            for b in range(NBUF):
                v = group * NBUF + b
                drain_fill(v, b)
                out_slice = out_hbm.at[pl.ds(base_row + v * CHUNK, CHUNK)]
                pltpu.async_copy(rows[b], out_slice, ssem[b]).wait()

                @pl.when(group + 1 < ngroups)
                def _():
                    fill(v + NBUF, b)
            return carry

        lax.fori_loop(0, ngroups, body, 0, unroll=False)

    return gather_kernel


def kernel(positions, table):
    b, l = positions.shape
    b_flat = b * l
    idx = positions.astype(jnp.int32).reshape(
        NUM_WORKERS, b_flat // (NUM_WORKERS * IDXC), IDXC
    )
    out = _make_gather(b_flat)(idx, table)
    return out.reshape(b, l, DIM)
